# baseline (speedup 1.0000x reference)
import jax
import jax.numpy as jnp
from jax import lax
from jax.experimental import pallas as pl
from jax.experimental.pallas import tpu as pltpu
from jax.experimental.pallas import tpu_sc as plsc

NUM_USERS = 25000
NUM_ITEMS = 25000
N_NODES = NUM_USERS + NUM_ITEMS
EMB_DIM = 64
HALF = 32
N_LAYERS = 3
N_EDGES = 800000

NC = 2
NS = 16
BLK = 128
EDGES_PAD = 819200
BLOCKS = EDGES_PAD // (NS * BLK)
N_PAD = 51200
STRIPE = N_PAD // NS
SEC = 2 * N_PAD
ZROWS = 128
NB_R = 4
NB_I = 8


def _lane_bcast(vec, e):
    idx = jnp.full((16, 1), e, dtype=jnp.int32)
    dn = lax.GatherDimensionNumbers(
        offset_dims=(), collapsed_slice_dims=(0,), start_index_map=(0,))
    return lax.gather(vec, idx, dn, slice_sizes=(1,),
                      mode=lax.GatherScatterMode.PROMISE_IN_BOUNDS)


def _gcn_body(e0h, srcp, dstp, wp,
              big_tab,
              acc, idx_s, idx_d, wv, rows, zbuf,
              sem_i, sem_g, sem_sc, sem_z):
    c = lax.axis_index("c")
    s = lax.axis_index("s")
    coff = c * N_PAD
    row0 = s * STRIPE
    ebase = s * (BLOCKS * BLK)

    def _zrow(r, cy):
        z16 = jnp.zeros((16,), jnp.float32)
        zbuf[r, pl.ds(0, 16)] = z16
        zbuf[r, pl.ds(16, 16)] = z16
        return cy
    lax.fori_loop(0, ZROWS, _zrow, 0)
    pltpu.sync_copy(e0h.at[pl.ds(coff + row0, STRIPE), :],
                    big_tab.at[pl.ds(coff + row0, STRIPE), :])

    def _zero_stripe():
        def _zi(k, cy):
            pltpu.async_copy(zbuf, acc.at[pl.ds(row0 + k * ZROWS, ZROWS), :],
                             sem_z)
            return cy
        lax.fori_loop(0, STRIPE // ZROWS, _zi, 0)

        def _zw(k, cy):
            pltpu.make_async_copy(
                zbuf, acc.at[pl.ds(row0, ZROWS), :], sem_z).wait()
            return cy
        lax.fori_loop(0, STRIPE // ZROWS, _zw, 0)
    _zero_stripe()
    plsc.subcore_barrier()

    def _issue_idx(ji, b):
        pltpu.async_copy(srcp.at[pl.ds(ebase + b * BLK, BLK)],
                         idx_s[ji], sem_i[ji])
        pltpu.async_copy(dstp.at[pl.ds(ebase + b * BLK, BLK)],
                         idx_d[ji], sem_i[ji])
        pltpu.async_copy(wp.at[pl.ds(ebase + b * BLK, BLK)],
                         wv[ji], sem_i[ji])

    def _wait_idx(ji):
        pltpu.make_async_copy(srcp.at[pl.ds(0, BLK)], idx_s[ji],
                              sem_i[ji]).wait()
        pltpu.make_async_copy(dstp.at[pl.ds(0, BLK)], idx_d[ji],
                              sem_i[ji]).wait()
        pltpu.make_async_copy(wp.at[pl.ds(0, BLK)], wv[ji],
                              sem_i[ji]).wait()

    def _issue_gather(jr, ji):
        pltpu.async_copy(big_tab.at[idx_s[ji]], rows[jr], sem_g[jr])

    def _wait_gather(jr, ji):
        pltpu.make_async_copy(big_tab.at[idx_s[ji]], rows[jr],
                              sem_g[jr]).wait()

    def _issue_scat(jr, ji):
        pltpu.async_copy(rows[jr], acc.at[idx_d[ji]], sem_sc[jr], add=True)

    def _wait_scat(jr, ji):
        pltpu.make_async_copy(rows[jr], acc.at[idx_d[ji]], sem_sc[jr]).wait()

    def _mult(jr, ji):
        def _grp(g, cy):
            wvec = wv[ji][pl.ds(g * 16, 16)]
            for e in range(16):
                r = g * 16 + e
                ws = _lane_bcast(wvec, e)
                rows[jr][r, pl.ds(0, 16)] = rows[jr][r, pl.ds(0, 16)] * ws
                rows[jr][r, pl.ds(16, 16)] = rows[jr][r, pl.ds(16, 16)] * ws
            return cy
        lax.fori_loop(0, BLK // 16, _grp, 0)

    def _layer(l, cy):
        loff = l * SEC + coff
        _issue_idx(0, 0)
        _issue_idx(1, 1)

        def _group(g, cy2):
            for j in range(NB_I):
                b = g * NB_I + j
                jr, ji = j % NB_R, j
                jrm1, jim1 = (j - 1) % NB_R, (j - 1) % NB_I
                _wait_idx(ji)
                idx_s[ji][...] = idx_s[ji][...] + loff
                if j >= NB_R:
                    _wait_scat(jr, ji)
                else:
                    @pl.when(g >= 1)
                    def _():
                        _wait_scat(jr, ji)
                _issue_gather(jr, ji)
                if j >= 1:
                    _wait_gather(jrm1, jim1)
                    _mult(jrm1, jim1)
                    _issue_scat(jrm1, jim1)
                else:
                    @pl.when(g >= 1)
                    def _():
                        _wait_gather(jrm1, jim1)
                        _mult(jrm1, jim1)
                        _issue_scat(jrm1, jim1)
                if j < NB_I - 2:
                    _issue_idx((j + 2) % NB_I, b + 2)
                else:
                    @pl.when(g < BLOCKS // NB_I - 1)
                    def _():
                        _issue_idx((j + 2) % NB_I, b + 2)
            return cy2
        lax.fori_loop(0, BLOCKS // NB_I, _group, 0)

        _wait_gather((BLOCKS - 1) % NB_R, (BLOCKS - 1) % NB_I)
        _mult((BLOCKS - 1) % NB_R, (BLOCKS - 1) % NB_I)
        _issue_scat((BLOCKS - 1) % NB_R, (BLOCKS - 1) % NB_I)
        for j in range(NB_R):
            _wait_scat(j, (BLOCKS - NB_R + j) % NB_I)
        plsc.subcore_barrier()

        dsec = (l + 1) * SEC + coff
        pltpu.sync_copy(acc.at[pl.ds(row0, STRIPE), :],
                        big_tab.at[pl.ds(dsec + row0, STRIPE), :])

        @pl.when(l < N_LAYERS - 1)
        def _():
            _zero_stripe()
        plsc.subcore_barrier()
        return cy
    lax.fori_loop(0, N_LAYERS, _layer, 0)


_gcn = pl.kernel(
    _gcn_body,
    out_type=jax.ShapeDtypeStruct((4 * SEC, HALF), jnp.float32),
    mesh=plsc.VectorSubcoreMesh(core_axis_name="c", subcore_axis_name="s"),
    compiler_params=pltpu.CompilerParams(use_tc_tiling_on_sc=False),
    scratch_types=[
        pltpu.VMEM_SHARED((N_PAD, HALF), jnp.float32),
        [pltpu.VMEM((BLK,), jnp.int32) for _ in range(NB_I)],
        [pltpu.VMEM((BLK,), jnp.int32) for _ in range(NB_I)],
        [pltpu.VMEM((BLK,), jnp.float32) for _ in range(NB_I)],
        [pltpu.VMEM((BLK, HALF), jnp.float32) for _ in range(NB_R)],
        pltpu.VMEM((ZROWS, HALF), jnp.float32),
        [pltpu.SemaphoreType.DMA for _ in range(NB_I)],
        [pltpu.SemaphoreType.DMA for _ in range(NB_R)],
        [pltpu.SemaphoreType.DMA for _ in range(NB_R)],
        pltpu.SemaphoreType.DMA,
    ],
)


def _mean_body(x_ref, o_ref):
    o_ref[...] = (x_ref[0] + x_ref[1] + x_ref[2] + x_ref[3]) * 0.25


_mean = pl.pallas_call(
    _mean_body,
    grid=(8,),
    in_specs=[pl.BlockSpec((4, 3200, 128), lambda i: (0, i, 0))],
    out_specs=pl.BlockSpec((3200, 128), lambda i: (i, 0)),
    out_shape=jax.ShapeDtypeStruct((25600, 128), jnp.float32),
)


def kernel(edge_index, edge_weight, user_emb, item_emb):
    src = edge_index[0]
    dst = edge_index[1]
    all_emb = jnp.concatenate([user_emb, item_emb], axis=0)
    rpad = N_PAD - N_NODES
    e0h = jnp.concatenate([
        jnp.pad(all_emb[:, :HALF], ((0, rpad), (0, 0))),
        jnp.pad(all_emb[:, HALF:], ((0, rpad), (0, 0))),
    ], axis=0)
    pad = EDGES_PAD - N_EDGES
    srcp = jnp.pad(src, (0, pad))
    dstp = jnp.pad(dst, (0, pad))
    wp = jnp.pad(edge_weight, (0, pad))
    big_tab = _gcn(e0h, srcp, dstp, wp)
    out_flat = _mean(big_tab.reshape(4, 25600, 128)).reshape(SEC, HALF)
    users = jnp.concatenate(
        [out_flat[:NUM_USERS], out_flat[N_PAD:N_PAD + NUM_USERS]], axis=1)
    items = jnp.concatenate(
        [out_flat[NUM_USERS:N_NODES], out_flat[N_PAD + NUM_USERS:N_PAD + N_NODES]],
        axis=1)
    return users, items


# speedup vs baseline: 1.3473x; 1.3473x over previous
import jax
import jax.numpy as jnp
from jax import lax
from jax.experimental import pallas as pl
from jax.experimental.pallas import tpu as pltpu
from jax.experimental.pallas import tpu_sc as plsc

NUM_USERS = 25000
NUM_ITEMS = 25000
N_NODES = NUM_USERS + NUM_ITEMS
EMB_DIM = 64
HALF = 32
N_LAYERS = 3
N_EDGES = 800000

NC = 2
NS = 16
BLK = 256
EDGES_PAD = 819200
BLOCKS = EDGES_PAD // (NS * BLK)
N_PAD = 51200
STRIPE = N_PAD // NS
SEC = 2 * N_PAD
ZROWS = 128
NB_R = 2
NB_I = 8


def _lane_bcast(vec, e):
    idx = jnp.full((16, 1), e, dtype=jnp.int32)
    dn = lax.GatherDimensionNumbers(
        offset_dims=(), collapsed_slice_dims=(0,), start_index_map=(0,))
    return lax.gather(vec, idx, dn, slice_sizes=(1,),
                      mode=lax.GatherScatterMode.PROMISE_IN_BOUNDS)


def _gcn_body(e0h, srcp, dstp, wp,
              big_tab,
              acc, idx_s, idx_d, wv, rows, zbuf,
              sem_i, sem_g, sem_sc, sem_z, sem_e):
    c = lax.axis_index("c")
    s = lax.axis_index("s")
    coff = c * N_PAD
    row0 = s * STRIPE
    ebase = s * (BLOCKS * BLK)

    def _zrow(r, cy):
        z16 = jnp.zeros((16,), jnp.float32)
        zbuf[r, pl.ds(0, 16)] = z16
        zbuf[r, pl.ds(16, 16)] = z16
        return cy
    lax.fori_loop(0, ZROWS, _zrow, 0)
    pltpu.async_copy(e0h.at[pl.ds(coff + row0, STRIPE), :],
                     big_tab.at[pl.ds(coff + row0, STRIPE), :], sem_e)

    def _zero_stripe():
        def _zi(k, cy):
            pltpu.async_copy(zbuf, acc.at[pl.ds(row0 + k * ZROWS, ZROWS), :],
                             sem_z)
            return cy
        lax.fori_loop(0, STRIPE // ZROWS, _zi, 0)

        def _zw(k, cy):
            pltpu.make_async_copy(
                zbuf, acc.at[pl.ds(row0, ZROWS), :], sem_z).wait()
            return cy
        lax.fori_loop(0, STRIPE // ZROWS, _zw, 0)
    _zero_stripe()
    plsc.subcore_barrier()

    def _issue_idx(ji, b):
        pltpu.async_copy(srcp.at[pl.ds(ebase + b * BLK, BLK)],
                         idx_s[ji], sem_i[ji])
        pltpu.async_copy(dstp.at[pl.ds(ebase + b * BLK, BLK)],
                         idx_d[ji], sem_i[ji])
        pltpu.async_copy(wp.at[pl.ds(ebase + b * BLK, BLK)],
                         wv[ji], sem_i[ji])

    def _wait_idx(ji):
        pltpu.make_async_copy(srcp.at[pl.ds(0, BLK)], idx_s[ji],
                              sem_i[ji]).wait()
        pltpu.make_async_copy(dstp.at[pl.ds(0, BLK)], idx_d[ji],
                              sem_i[ji]).wait()
        pltpu.make_async_copy(wp.at[pl.ds(0, BLK)], wv[ji],
                              sem_i[ji]).wait()

    def _issue_scat(jr, ji):
        pltpu.async_copy(rows[jr], acc.at[idx_d[ji]], sem_sc[jr], add=True)

    def _wait_scat(jr, ji):
        pltpu.make_async_copy(rows[jr], acc.at[idx_d[ji]], sem_sc[jr]).wait()

    def _mult(jr, ji):
        def _grp(g, cy):
            wvec = wv[ji][pl.ds(g * 16, 16)]
            for e in range(16):
                r = g * 16 + e
                ws = _lane_bcast(wvec, e)
                rows[jr][r, pl.ds(0, 16)] = rows[jr][r, pl.ds(0, 16)] * ws
                rows[jr][r, pl.ds(16, 16)] = rows[jr][r, pl.ds(16, 16)] * ws
            return cy
        lax.fori_loop(0, BLK // 16, _grp, 0)

    def _layer(l, cy):
        loff = l * SEC + coff
        _issue_idx(0, 0)
        _issue_idx(1, 1)

        def _issue_gather(jr, ji):
            @pl.when(l == 0)
            def _():
                pltpu.async_copy(e0h.at[idx_s[ji]], rows[jr], sem_g[jr])
            @pl.when(l > 0)
            def _():
                pltpu.async_copy(big_tab.at[idx_s[ji]], rows[jr], sem_g[jr])

        def _wait_gather(jr, ji):
            @pl.when(l == 0)
            def _():
                pltpu.make_async_copy(e0h.at[idx_s[ji]], rows[jr],
                                      sem_g[jr]).wait()
            @pl.when(l > 0)
            def _():
                pltpu.make_async_copy(big_tab.at[idx_s[ji]], rows[jr],
                                      sem_g[jr]).wait()

        def _group(g, cy2):
            for j in range(NB_I):
                b = g * NB_I + j
                jr, ji = j % NB_R, j
                jrm1, jim1 = (j - 1) % NB_R, (j - 1) % NB_I
                _wait_idx(ji)
                idx_s[ji][...] = idx_s[ji][...] + loff
                if j >= NB_R:
                    _wait_scat(jr, ji)
                else:
                    @pl.when(g >= 1)
                    def _():
                        _wait_scat(jr, ji)
                _issue_gather(jr, ji)
                if j >= 1:
                    _wait_gather(jrm1, jim1)
                    _mult(jrm1, jim1)
                    _issue_scat(jrm1, jim1)
                else:
                    @pl.when(g >= 1)
                    def _():
                        _wait_gather(jrm1, jim1)
                        _mult(jrm1, jim1)
                        _issue_scat(jrm1, jim1)
                if j < NB_I - 2:
                    _issue_idx((j + 2) % NB_I, b + 2)
                else:
                    @pl.when(g < BLOCKS // NB_I - 1)
                    def _():
                        _issue_idx((j + 2) % NB_I, b + 2)
            return cy2
        lax.fori_loop(0, BLOCKS // NB_I, _group, 0)

        _wait_gather((BLOCKS - 1) % NB_R, (BLOCKS - 1) % NB_I)
        _mult((BLOCKS - 1) % NB_R, (BLOCKS - 1) % NB_I)
        _issue_scat((BLOCKS - 1) % NB_R, (BLOCKS - 1) % NB_I)
        for j in range(NB_R):
            _wait_scat(j, (BLOCKS - NB_R + j) % NB_I)
        plsc.subcore_barrier()

        dsec = (l + 1) * SEC + coff
        pltpu.sync_copy(acc.at[pl.ds(row0, STRIPE), :],
                        big_tab.at[pl.ds(dsec + row0, STRIPE), :])

        @pl.when(l < N_LAYERS - 1)
        def _():
            _zero_stripe()
        plsc.subcore_barrier()
        return cy
    lax.fori_loop(0, N_LAYERS, _layer, 0)
    pltpu.make_async_copy(e0h.at[pl.ds(coff + row0, STRIPE), :],
                          big_tab.at[pl.ds(coff + row0, STRIPE), :],
                          sem_e).wait()


_gcn = pl.kernel(
    _gcn_body,
    out_type=jax.ShapeDtypeStruct((4 * SEC, HALF), jnp.float32),
    mesh=plsc.VectorSubcoreMesh(core_axis_name="c", subcore_axis_name="s"),
    compiler_params=pltpu.CompilerParams(use_tc_tiling_on_sc=False),
    scratch_types=[
        pltpu.VMEM_SHARED((N_PAD, HALF), jnp.float32),
        [pltpu.VMEM((BLK,), jnp.int32) for _ in range(NB_I)],
        [pltpu.VMEM((BLK,), jnp.int32) for _ in range(NB_I)],
        [pltpu.VMEM((BLK,), jnp.float32) for _ in range(NB_I)],
        [pltpu.VMEM((BLK, HALF), jnp.float32) for _ in range(NB_R)],
        pltpu.VMEM((ZROWS, HALF), jnp.float32),
        [pltpu.SemaphoreType.DMA for _ in range(NB_I)],
        [pltpu.SemaphoreType.DMA for _ in range(NB_R)],
        [pltpu.SemaphoreType.DMA for _ in range(NB_R)],
        pltpu.SemaphoreType.DMA,
        pltpu.SemaphoreType.DMA,
    ],
)


def _mean_body(x_ref, o_ref):
    o_ref[...] = (x_ref[0] + x_ref[1] + x_ref[2] + x_ref[3]) * 0.25


_mean = pl.pallas_call(
    _mean_body,
    grid=(8,),
    in_specs=[pl.BlockSpec((4, 3200, 128), lambda i: (0, i, 0))],
    out_specs=pl.BlockSpec((3200, 128), lambda i: (i, 0)),
    out_shape=jax.ShapeDtypeStruct((25600, 128), jnp.float32),
)


def kernel(edge_index, edge_weight, user_emb, item_emb):
    src = edge_index[0]
    dst = edge_index[1]
    all_emb = jnp.concatenate([user_emb, item_emb], axis=0)
    rpad = N_PAD - N_NODES
    e0h = jnp.concatenate([
        jnp.pad(all_emb[:, :HALF], ((0, rpad), (0, 0))),
        jnp.pad(all_emb[:, HALF:], ((0, rpad), (0, 0))),
    ], axis=0)
    pad = EDGES_PAD - N_EDGES
    srcp = jnp.pad(src, (0, pad))
    dstp = jnp.pad(dst, (0, pad))
    wp = jnp.pad(edge_weight, (0, pad))
    big_tab = _gcn(e0h, srcp, dstp, wp)
    out_flat = _mean(big_tab.reshape(4, 25600, 128)).reshape(SEC, HALF)
    users = jnp.concatenate(
        [out_flat[:NUM_USERS], out_flat[N_PAD:N_PAD + NUM_USERS]], axis=1)
    items = jnp.concatenate(
        [out_flat[NUM_USERS:N_NODES], out_flat[N_PAD + NUM_USERS:N_PAD + N_NODES]],
        axis=1)
    return users, items
